# baseline (speedup 1.0000x reference)
import functools

import jax
import jax.numpy as jnp
from jax import lax
from jax.experimental import pallas as pl
from jax.experimental.pallas import tpu as pltpu
from jax.experimental.pallas import tpu_sc as plsc

_CH = 125
_G = 64
_RB = 2000


def _fill_rows(ref, nrows, ncols, value):
    v = jnp.full((16,), value, dtype=jnp.float32)

    def body(j, c):
        for kk in range(ncols // 16):
            ref[j, pl.ds(kk * 16, 16)] = v
        return c

    lax.fori_loop(0, nrows, body, 0)


def _round_up(a, b):
    return -(-a // b) * b


@functools.cache
def _make_deg(N, E, NC, NS):
    NW = NC * NS
    NCH = E // (NW * _CH)
    RPT = _round_up(N, 16 * NS) // NS
    NPAD = RPT * NS
    mesh = plsc.VectorSubcoreMesh(core_axis_name="c", subcore_axis_name="s")

    @functools.partial(
        pl.kernel,
        out_type=jax.ShapeDtypeStruct((NC, NPAD, 16), jnp.float32),
        mesh=mesh,
        compiler_params=pltpu.CompilerParams(use_tc_tiling_on_sc=False),
        scratch_types=[
            pltpu.VMEM((NCH, _CH), jnp.int32),
            pltpu.VMEM((_CH, 16), jnp.float32),
            pltpu.VMEM((RPT, 16), jnp.float32),
            pltpu.SemaphoreType.DMA,
            pltpu.VMEM_SHARED((NPAD, 16), jnp.float32),
        ],
    )
    def k(e4_hbm, out_hbm, dstv, onesv, zbuf, ssem, acc):
        c = lax.axis_index("c")
        s = lax.axis_index("s")
        wid = s * NC + c
        _fill_rows(onesv, _CH, 16, 1.0)
        _fill_rows(zbuf, RPT, 16, 0.0)
        pltpu.sync_copy(zbuf, acc.at[pl.ds(s * RPT, RPT)])
        pltpu.sync_copy(e4_hbm.at[1, wid], dstv)
        plsc.subcore_barrier()

        def step(kk, carry):
            ds_ = [pltpu.async_copy(onesv, acc.at[dstv.at[8 * kk + b]],
                                    ssem, add=True) for b in range(8)]
            for d in ds_:
                d.wait()
            return carry

        lax.fori_loop(0, NCH // 8, step, 0)
        for j in range(NCH - NCH % 8, NCH):
            pltpu.sync_copy(onesv, acc.at[dstv.at[j]], add=True)
        plsc.subcore_barrier()
        pltpu.sync_copy(acc.at[pl.ds(s * RPT, RPT)],
                        out_hbm.at[c, pl.ds(s * RPT, RPT)])

    return k


@functools.cache
def _make_msg(N, E, D, NC, NS):
    NW = NC * NS
    NCH = E // (NW * _CH)
    RPT = _round_up(N, 16 * NS) // NS
    NPAD = RPT * NS
    mesh = plsc.VectorSubcoreMesh(core_axis_name="c", subcore_axis_name="s")

    KB = 4

    @functools.partial(
        pl.kernel,
        out_type=jax.ShapeDtypeStruct((NC, NPAD, D), jnp.float32),
        mesh=mesh,
        compiler_params=pltpu.CompilerParams(use_tc_tiling_on_sc=False),
        scratch_types=[
            pltpu.VMEM((NCH, _CH), jnp.int32),
            pltpu.VMEM((NCH, _CH), jnp.int32),
            pltpu.VMEM((2 * KB, _CH, D), jnp.float32),
            pltpu.SemaphoreType.DMA,
            pltpu.SemaphoreType.DMA,
            pltpu.SemaphoreType.DMA,
            pltpu.SemaphoreType.DMA,
            pltpu.VMEM_SHARED((NPAD, D), jnp.float32),
        ],
    )
    def k(t_hbm, e4_hbm, out_hbm,
          srcv, dstv, rows, gsA, gsB, ssA, ssB, acc):
        c = lax.axis_index("c")
        s = lax.axis_index("s")
        wid = s * NC + c
        ZR = RPT // 8
        zv = jnp.zeros((16,), jnp.float32)

        def zfill(j, carry):
            for kk in range(D // 16):
                rows[0, j, pl.ds(kk * 16, 16)] = zv
            return carry

        lax.fori_loop(0, ZR, zfill, 0)
        for t in range(8):
            pltpu.sync_copy(rows.at[0, pl.ds(0, ZR)],
                            acc.at[pl.ds(s * RPT + t * ZR, ZR)])
        pltpu.sync_copy(e4_hbm.at[0, wid], srcv)
        pltpu.sync_copy(e4_hbm.at[1, wid], dstv)
        plsc.subcore_barrier()

        def fire_gathers(bank, base, sem):
            for b in range(KB):
                pltpu.async_copy(t_hbm.at[srcv.at[base + b]],
                                 rows.at[bank * KB + b], sem)

        def wait_gathers(bank, sem):
            for b in range(KB):
                pltpu.make_async_copy(t_hbm.at[srcv.at[0]],
                                      rows.at[bank * KB + b], sem).wait()

        def scatter_bank(bank, base, sem):
            ds_ = [pltpu.async_copy(rows.at[bank * KB + b],
                                    acc.at[dstv.at[base + b]], sem, add=True)
                   for b in range(KB)]
            for d in ds_:
                d.wait()

        NCH_B = (NCH // (2 * KB)) * 2 * KB
        fire_gathers(0, 0, gsA)

        def step(kk, carry):
            base = 2 * KB * kk
            fire_gathers(1, base + KB, gsB)
            wait_gathers(0, gsA)
            scatter_bank(0, base, ssA)

            @pl.when(base + 3 * KB <= NCH_B)
            def _():
                fire_gathers(0, base + 2 * KB, gsA)

            wait_gathers(1, gsB)
            scatter_bank(1, base + KB, ssB)
            return carry

        lax.fori_loop(0, NCH // (2 * KB), step, 0)
        for j in range(NCH - NCH % (2 * KB), NCH):
            pltpu.async_copy(t_hbm.at[srcv.at[j]], rows.at[0], gsA).wait()
            pltpu.async_copy(rows.at[0], acc.at[dstv.at[j]], gsA,
                             add=True).wait()
        plsc.subcore_barrier()
        pltpu.sync_copy(acc.at[pl.ds(s * RPT, RPT)],
                        out_hbm.at[c, pl.ds(s * RPT, RPT)])

    return k


def _stage1a_call(x, W1):
    N, Din = x.shape
    D1 = W1.shape[1]
    NB = N // _RB

    def body(x_ref, w_ref, xw_ref):
        xw_ref[...] = jnp.dot(x_ref[...], w_ref[...],
                              preferred_element_type=jnp.float32)

    return pl.pallas_call(
        body,
        grid=(NB,),
        in_specs=[
            pl.BlockSpec((_RB, Din), lambda i: (i, 0)),
            pl.BlockSpec((Din, D1), lambda i: (0, 0)),
        ],
        out_specs=pl.BlockSpec((_RB, D1), lambda i: (i, 0)),
        out_shape=jax.ShapeDtypeStruct((N, D1), jnp.float32),
    )(x, W1)


def _stage1b_call(xw, degp, NC):
    N, D1 = xw.shape
    NB = N // _RB

    def body(xw_ref, dp_ref, sc_ref, dis_ref):
        deg = sum(dp_ref[i, :, 0:1] for i in range(NC))
        dis = lax.rsqrt(deg)
        sc_ref[...] = xw_ref[...] * dis
        dis_ref[...] = dis

    return pl.pallas_call(
        body,
        grid=(NB,),
        in_specs=[
            pl.BlockSpec((_RB, D1), lambda i: (i, 0)),
            pl.BlockSpec((NC, _RB, 16), lambda i: (0, i, 0)),
        ],
        out_specs=[
            pl.BlockSpec((_RB, D1), lambda i: (i, 0)),
            pl.BlockSpec((_RB, 1), lambda i: (i, 0)),
        ],
        out_shape=[
            jax.ShapeDtypeStruct((N, D1), jnp.float32),
            jax.ShapeDtypeStruct((N, 1), jnp.float32),
        ],
    )(xw, degp)


def _stage2_call(acc1, dis, b1r, W2, b2r, NC):
    N = dis.shape[0]
    D1 = W2.shape[0]
    D2 = W2.shape[1]
    NB = N // _RB

    def body(a_ref, dis_ref, b1_ref, w_ref, sc_ref):
        accsum = sum(a_ref[i] for i in range(NC))
        dis = dis_ref[...]
        h1 = jnp.maximum(dis * accsum + b1_ref[...], 0.0)
        xw2 = jnp.dot(h1, w_ref[...], preferred_element_type=jnp.float32)
        sc_ref[...] = xw2 * dis

    return pl.pallas_call(
        body,
        grid=(NB,),
        in_specs=[
            pl.BlockSpec((NC, _RB, D1), lambda i: (0, i, 0)),
            pl.BlockSpec((_RB, 1), lambda i: (i, 0)),
            pl.BlockSpec((1, D1), lambda i: (0, 0)),
            pl.BlockSpec((D1, D2), lambda i: (0, 0)),
        ],
        out_specs=pl.BlockSpec((_RB, D2), lambda i: (i, 0)),
        out_shape=jax.ShapeDtypeStruct((N, D2), jnp.float32),
    )(acc1, dis, b1r, W2)


def _stage3_call(acc2, dis, b2r, batch_row, Wf1, bf1r, Wf2, bf2r,
                 Wf3, bf3r, NC):
    N = dis.shape[0]
    D2 = b2r.shape[1]
    F1 = Wf1.shape[1]
    F2 = Wf2.shape[1]
    NB = N // _RB

    def body(a_ref, dis_ref, b2_ref, b_ref, wf1_ref, bf1_ref,
             wf2_ref, bf2_ref, wf3_ref, bf3_ref, out_ref, sum_acc, cnt_acc):
        i = pl.program_id(0)
        accsum = sum(a_ref[k] for k in range(NC))
        h2 = jnp.maximum(dis_ref[...] * accsum + b2_ref[...], 0.0)
        seg = b_ref[0]
        gids = lax.broadcasted_iota(jnp.int32, (_G, _RB), 0)
        pt = (gids == seg).astype(jnp.float32)
        part = jnp.dot(pt, h2, preferred_element_type=jnp.float32)
        cnt = jnp.dot(pt, jnp.ones((_RB, 1), jnp.float32),
                      preferred_element_type=jnp.float32)

        @pl.when(i == 0)
        def _():
            sum_acc[...] = part
            cnt_acc[...] = cnt

        @pl.when(i > 0)
        def _():
            sum_acc[...] += part
            cnt_acc[...] += cnt

        @pl.when(i == NB - 1)
        def _():
            pooled = sum_acc[...] / jnp.maximum(cnt_acc[...], 1.0)
            hh = jnp.maximum(
                jnp.dot(pooled, wf1_ref[...],
                        preferred_element_type=jnp.float32) + bf1_ref[...], 0.0)
            hh = jnp.maximum(
                jnp.dot(hh, wf2_ref[...],
                        preferred_element_type=jnp.float32) + bf2_ref[...], 0.0)
            out_ref[...] = (jnp.dot(hh, wf3_ref[...],
                                    preferred_element_type=jnp.float32)
                            + bf3_ref[...])

    return pl.pallas_call(
        body,
        grid=(NB,),
        in_specs=[
            pl.BlockSpec((NC, _RB, D2), lambda i: (0, i, 0)),
            pl.BlockSpec((_RB, 1), lambda i: (i, 0)),
            pl.BlockSpec((1, D2), lambda i: (0, 0)),
            pl.BlockSpec((1, 1, _RB), lambda i: (i, 0, 0)),
            pl.BlockSpec((D2, F1), lambda i: (0, 0)),
            pl.BlockSpec((1, F1), lambda i: (0, 0)),
            pl.BlockSpec((F1, F2), lambda i: (0, 0)),
            pl.BlockSpec((1, F2), lambda i: (0, 0)),
            pl.BlockSpec((F2, 1), lambda i: (0, 0)),
            pl.BlockSpec((1, 1), lambda i: (0, 0)),
        ],
        out_specs=pl.BlockSpec((_G, 1), lambda i: (0, 0)),
        out_shape=jax.ShapeDtypeStruct((_G, 1), jnp.float32),
        scratch_shapes=[
            pltpu.VMEM((_G, D2), jnp.float32),
            pltpu.VMEM((_G, 1), jnp.float32),
        ],
    )(acc2, dis, b2r, batch_row, Wf1, bf1r, Wf2, bf2r, Wf3, bf3r)


def kernel(x, edge_index, batch, W1, b1, W2, b2, Wf1, bf1, Wf2, bf2, Wf3, bf3):
    N, Din = x.shape
    E = edge_index.shape[1]
    info = plsc.get_sparse_core_info()
    NC, NS = info.num_cores, info.num_subcores
    NW = NC * NS
    assert N % _RB == 0

    EPT = NW * _CH
    EX = _round_up(E + N, EPT)
    loop = jnp.arange(N, dtype=edge_index.dtype)
    pad_src = jnp.zeros((EX - E - N,), edge_index.dtype)
    pad_dst = jnp.full((EX - E - N,), N, edge_index.dtype)
    srcx = jnp.concatenate([edge_index[0], loop, pad_src])
    dstx = jnp.concatenate([edge_index[1], loop, pad_dst])
    e4 = jnp.stack([srcx, dstx]).reshape(2, NW, EX // (NW * _CH), _CH)

    batch_row = batch.reshape(N // _RB, 1, _RB)
    b1r = b1.reshape(1, -1)
    b2r = b2.reshape(1, -1)
    bf1r = bf1.reshape(1, -1)
    bf2r = bf2.reshape(1, -1)
    bf3r = bf3.reshape(1, -1)

    degp = _make_deg(N, EX, NC, NS)(e4)
    xw1 = _stage1a_call(x, W1)
    scaled1, dis = _stage1b_call(xw1, degp, NC)
    acc1 = _make_msg(N, EX, W1.shape[1], NC, NS)(scaled1, e4)
    scaled2 = _stage2_call(acc1, dis, b1r, W2, b2r, NC)
    acc2 = _make_msg(N, EX, W2.shape[1], NC, NS)(scaled2, e4)
    return _stage3_call(acc2, dis, b2r, batch_row, Wf1, bf1r, Wf2, bf2r,
                        Wf3, bf3r, NC)


# speedup vs baseline: 1.4560x; 1.4560x over previous
import functools

import jax
import jax.numpy as jnp
from jax import lax
from jax.experimental import pallas as pl
from jax.experimental.pallas import tpu as pltpu
from jax.experimental.pallas import tpu_sc as plsc

_CH = 125
_G = 64
_RB = 2000


def _fill_rows(ref, nrows, ncols, value):
    v = jnp.full((16,), value, dtype=jnp.float32)

    def body(j, c):
        for kk in range(ncols // 16):
            ref[j, pl.ds(kk * 16, 16)] = v
        return c

    lax.fori_loop(0, nrows, body, 0)


def _round_up(a, b):
    return -(-a // b) * b


@functools.cache
def _make_deg(N, E, NC, NS):
    NW = NC * NS
    NCH = E // (NW * _CH)
    RPT = _round_up(N, 16 * NS) // NS
    NPAD = RPT * NS
    mesh = plsc.VectorSubcoreMesh(core_axis_name="c", subcore_axis_name="s")

    @functools.partial(
        pl.kernel,
        out_type=jax.ShapeDtypeStruct((NC, NPAD, 16), jnp.float32),
        mesh=mesh,
        compiler_params=pltpu.CompilerParams(use_tc_tiling_on_sc=False),
        scratch_types=[
            pltpu.VMEM((NCH, _CH), jnp.int32),
            pltpu.VMEM((_CH, 16), jnp.float32),
            pltpu.VMEM((RPT, 16), jnp.float32),
            pltpu.SemaphoreType.DMA,
            pltpu.VMEM_SHARED((NPAD, 16), jnp.float32),
        ],
    )
    def k(e4_hbm, out_hbm, dstv, onesv, zbuf, ssem, acc):
        c = lax.axis_index("c")
        s = lax.axis_index("s")
        wid = s * NC + c
        _fill_rows(onesv, _CH, 16, 1.0)
        _fill_rows(zbuf, RPT, 16, 0.0)
        pltpu.sync_copy(zbuf, acc.at[pl.ds(s * RPT, RPT)])
        pltpu.sync_copy(e4_hbm.at[1, wid], dstv)
        plsc.subcore_barrier()

        def step(kk, carry):
            ds_ = [pltpu.async_copy(onesv, acc.at[dstv.at[8 * kk + b]],
                                    ssem, add=True) for b in range(8)]
            for d in ds_:
                d.wait()
            return carry

        lax.fori_loop(0, NCH // 8, step, 0)
        for j in range(NCH - NCH % 8, NCH):
            pltpu.sync_copy(onesv, acc.at[dstv.at[j]], add=True)
        plsc.subcore_barrier()
        pltpu.sync_copy(acc.at[pl.ds(s * RPT, RPT)],
                        out_hbm.at[c, pl.ds(s * RPT, RPT)])

    return k


@functools.cache
def _make_msg(N, E, D, NC, NS):
    NW = NC * NS
    NCH = E // (NW * _CH)
    RPT = _round_up(N, 16 * NS) // NS
    NPAD = RPT * NS
    mesh = plsc.VectorSubcoreMesh(core_axis_name="c", subcore_axis_name="s")

    KB = 4

    @functools.partial(
        pl.kernel,
        out_type=jax.ShapeDtypeStruct((NC, NPAD, D), jnp.float32),
        mesh=mesh,
        compiler_params=pltpu.CompilerParams(use_tc_tiling_on_sc=False),
        scratch_types=[
            pltpu.VMEM((NCH, _CH), jnp.int32),
            pltpu.VMEM((NCH, _CH), jnp.int32),
            pltpu.VMEM((2 * KB, _CH, D), jnp.float32),
            pltpu.SemaphoreType.DMA,
            pltpu.SemaphoreType.DMA,
            pltpu.SemaphoreType.DMA,
            pltpu.SemaphoreType.DMA,
            pltpu.VMEM_SHARED((NPAD, D), jnp.float32),
        ],
    )
    def k(t_hbm, e4_hbm, out_hbm,
          srcv, dstv, rows, gsA, gsB, ssA, ssB, acc):
        c = lax.axis_index("c")
        s = lax.axis_index("s")
        wid = s * NC + c
        ZR = RPT // 8
        zv = jnp.zeros((16,), jnp.float32)

        def zfill(j, carry):
            for kk in range(D // 16):
                rows[0, j, pl.ds(kk * 16, 16)] = zv
            return carry

        lax.fori_loop(0, ZR, zfill, 0)
        for t in range(8):
            pltpu.sync_copy(rows.at[0, pl.ds(0, ZR)],
                            acc.at[pl.ds(s * RPT + t * ZR, ZR)])
        pltpu.sync_copy(e4_hbm.at[0, wid], srcv)
        pltpu.sync_copy(e4_hbm.at[1, wid], dstv)
        plsc.subcore_barrier()

        def fire_gathers(bank, base, sem):
            for b in range(KB):
                pltpu.async_copy(t_hbm.at[srcv.at[base + b]],
                                 rows.at[bank * KB + b], sem)

        def wait_gathers(bank, sem):
            for b in range(KB):
                pltpu.make_async_copy(t_hbm.at[srcv.at[0]],
                                      rows.at[bank * KB + b], sem).wait()

        def scatter_bank(bank, base, sem):
            ds_ = [pltpu.async_copy(rows.at[bank * KB + b],
                                    acc.at[dstv.at[base + b]], sem, add=True)
                   for b in range(KB)]
            for d in ds_:
                d.wait()

        NCH_B = (NCH // (2 * KB)) * 2 * KB
        fire_gathers(0, 0, gsA)

        def step(kk, carry):
            base = 2 * KB * kk
            fire_gathers(1, base + KB, gsB)
            wait_gathers(0, gsA)
            scatter_bank(0, base, ssA)

            @pl.when(base + 3 * KB <= NCH_B)
            def _():
                fire_gathers(0, base + 2 * KB, gsA)

            wait_gathers(1, gsB)
            scatter_bank(1, base + KB, ssB)
            return carry

        lax.fori_loop(0, NCH // (2 * KB), step, 0)
        for j in range(NCH - NCH % (2 * KB), NCH):
            pltpu.async_copy(t_hbm.at[srcv.at[j]], rows.at[0], gsA).wait()
            pltpu.async_copy(rows.at[0], acc.at[dstv.at[j]], gsA,
                             add=True).wait()
        plsc.subcore_barrier()
        pltpu.sync_copy(acc.at[pl.ds(s * RPT, RPT)],
                        out_hbm.at[c, pl.ds(s * RPT, RPT)])

    return k


def _stage1a_call(x, W1):
    N, Din = x.shape
    D1 = W1.shape[1]
    NB = N // _RB

    def body(x_ref, w_ref, xw_ref):
        xw_ref[...] = jnp.dot(x_ref[...], w_ref[...],
                              preferred_element_type=jnp.float32)

    return pl.pallas_call(
        body,
        grid=(NB,),
        in_specs=[
            pl.BlockSpec((_RB, Din), lambda i: (i, 0)),
            pl.BlockSpec((Din, D1), lambda i: (0, 0)),
        ],
        out_specs=pl.BlockSpec((_RB, D1), lambda i: (i, 0)),
        out_shape=jax.ShapeDtypeStruct((N, D1), jnp.float32),
    )(x, W1)


def _stage1b_call(xw, degp, b1r, NC):
    N, D1 = xw.shape
    NB = N // _RB

    def body(xw_ref, dp_ref, b_ref, sc_ref, sb_ref, dis_ref, inv_ref):
        deg = 1.0 + sum(dp_ref[i, :, 0:1] for i in range(NC))
        dis = lax.rsqrt(deg)
        inv = 1.0 / deg
        xw = xw_ref[...]
        sc_ref[...] = xw * dis
        sb_ref[...] = xw * inv + b_ref[...]
        dis_ref[...] = dis
        inv_ref[...] = inv

    return pl.pallas_call(
        body,
        grid=(NB,),
        in_specs=[
            pl.BlockSpec((_RB, D1), lambda i: (i, 0)),
            pl.BlockSpec((NC, _RB, 16), lambda i: (0, i, 0)),
            pl.BlockSpec((1, D1), lambda i: (0, 0)),
        ],
        out_specs=[
            pl.BlockSpec((_RB, D1), lambda i: (i, 0)),
            pl.BlockSpec((_RB, D1), lambda i: (i, 0)),
            pl.BlockSpec((_RB, 1), lambda i: (i, 0)),
            pl.BlockSpec((_RB, 1), lambda i: (i, 0)),
        ],
        out_shape=[
            jax.ShapeDtypeStruct((N, D1), jnp.float32),
            jax.ShapeDtypeStruct((N, D1), jnp.float32),
            jax.ShapeDtypeStruct((N, 1), jnp.float32),
            jax.ShapeDtypeStruct((N, 1), jnp.float32),
        ],
    )(xw, degp, b1r)


def _stage2_call(acc1, dis, inv, selfb1, W2, b2r, NC):
    N = dis.shape[0]
    D1 = selfb1.shape[1]
    D2 = W2.shape[1]
    NB = N // _RB

    def body(a_ref, dis_ref, inv_ref, sb1_ref, w_ref, b_ref,
             sc_ref, sb2_ref):
        accsum = sum(a_ref[i] for i in range(NC))
        dis = dis_ref[...]
        h1 = jnp.maximum(dis * accsum + sb1_ref[...], 0.0)
        xw2 = jnp.dot(h1, w_ref[...], preferred_element_type=jnp.float32)
        sc_ref[...] = xw2 * dis
        sb2_ref[...] = xw2 * inv_ref[...] + b_ref[...]

    return pl.pallas_call(
        body,
        grid=(NB,),
        in_specs=[
            pl.BlockSpec((NC, _RB, D1), lambda i: (0, i, 0)),
            pl.BlockSpec((_RB, 1), lambda i: (i, 0)),
            pl.BlockSpec((_RB, 1), lambda i: (i, 0)),
            pl.BlockSpec((_RB, D1), lambda i: (i, 0)),
            pl.BlockSpec((D1, D2), lambda i: (0, 0)),
            pl.BlockSpec((1, D2), lambda i: (0, 0)),
        ],
        out_specs=[
            pl.BlockSpec((_RB, D2), lambda i: (i, 0)),
            pl.BlockSpec((_RB, D2), lambda i: (i, 0)),
        ],
        out_shape=[
            jax.ShapeDtypeStruct((N, D2), jnp.float32),
            jax.ShapeDtypeStruct((N, D2), jnp.float32),
        ],
    )(acc1, dis, inv, selfb1, W2, b2r)


def _stage3_call(acc2, dis, selfb2, batch_row, Wf1, bf1r, Wf2, bf2r,
                 Wf3, bf3r, NC):
    N = dis.shape[0]
    D2 = selfb2.shape[1]
    F1 = Wf1.shape[1]
    F2 = Wf2.shape[1]
    NB = N // _RB

    def body(a_ref, dis_ref, sb2_ref, b_ref, wf1_ref, bf1_ref,
             wf2_ref, bf2_ref, wf3_ref, bf3_ref, out_ref, sum_acc, cnt_acc):
        i = pl.program_id(0)
        accsum = sum(a_ref[k] for k in range(NC))
        h2 = jnp.maximum(dis_ref[...] * accsum + sb2_ref[...], 0.0)
        seg = b_ref[0]
        gids = lax.broadcasted_iota(jnp.int32, (_G, _RB), 0)
        pt = (gids == seg).astype(jnp.float32)
        part = jnp.dot(pt, h2, preferred_element_type=jnp.float32)
        cnt = jnp.dot(pt, jnp.ones((_RB, 1), jnp.float32),
                      preferred_element_type=jnp.float32)

        @pl.when(i == 0)
        def _():
            sum_acc[...] = part
            cnt_acc[...] = cnt

        @pl.when(i > 0)
        def _():
            sum_acc[...] += part
            cnt_acc[...] += cnt

        @pl.when(i == NB - 1)
        def _():
            pooled = sum_acc[...] / jnp.maximum(cnt_acc[...], 1.0)
            hh = jnp.maximum(
                jnp.dot(pooled, wf1_ref[...],
                        preferred_element_type=jnp.float32) + bf1_ref[...], 0.0)
            hh = jnp.maximum(
                jnp.dot(hh, wf2_ref[...],
                        preferred_element_type=jnp.float32) + bf2_ref[...], 0.0)
            out_ref[...] = (jnp.dot(hh, wf3_ref[...],
                                    preferred_element_type=jnp.float32)
                            + bf3_ref[...])

    return pl.pallas_call(
        body,
        grid=(NB,),
        in_specs=[
            pl.BlockSpec((NC, _RB, D2), lambda i: (0, i, 0)),
            pl.BlockSpec((_RB, 1), lambda i: (i, 0)),
            pl.BlockSpec((_RB, D2), lambda i: (i, 0)),
            pl.BlockSpec((1, 1, _RB), lambda i: (i, 0, 0)),
            pl.BlockSpec((D2, F1), lambda i: (0, 0)),
            pl.BlockSpec((1, F1), lambda i: (0, 0)),
            pl.BlockSpec((F1, F2), lambda i: (0, 0)),
            pl.BlockSpec((1, F2), lambda i: (0, 0)),
            pl.BlockSpec((F2, 1), lambda i: (0, 0)),
            pl.BlockSpec((1, 1), lambda i: (0, 0)),
        ],
        out_specs=pl.BlockSpec((_G, 1), lambda i: (0, 0)),
        out_shape=jax.ShapeDtypeStruct((_G, 1), jnp.float32),
        scratch_shapes=[
            pltpu.VMEM((_G, D2), jnp.float32),
            pltpu.VMEM((_G, 1), jnp.float32),
        ],
    )(acc2, dis, selfb2, batch_row, Wf1, bf1r, Wf2, bf2r, Wf3, bf3r)


def kernel(x, edge_index, batch, W1, b1, W2, b2, Wf1, bf1, Wf2, bf2, Wf3, bf3):
    N, Din = x.shape
    E = edge_index.shape[1]
    info = plsc.get_sparse_core_info()
    NC, NS = info.num_cores, info.num_subcores
    NW = NC * NS
    assert N % _RB == 0

    assert E % (NW * _CH) == 0
    e4 = edge_index.reshape(2, NW, E // (NW * _CH), _CH)
    batch_row = batch.reshape(N // _RB, 1, _RB)
    b1r = b1.reshape(1, -1)
    b2r = b2.reshape(1, -1)
    bf1r = bf1.reshape(1, -1)
    bf2r = bf2.reshape(1, -1)
    bf3r = bf3.reshape(1, -1)

    degp = _make_deg(N, E, NC, NS)(e4)
    xw1 = _stage1a_call(x, W1)
    scaled1, selfb1, dis, inv = _stage1b_call(xw1, degp, b1r, NC)
    acc1 = _make_msg(N, E, W1.shape[1], NC, NS)(scaled1, e4)
    scaled2, selfb2 = _stage2_call(acc1, dis, inv, selfb1, W2, b2r, NC)
    acc2 = _make_msg(N, E, W2.shape[1], NC, NS)(scaled2, e4)
    return _stage3_call(acc2, dis, selfb2, batch_row, Wf1, bf1r, Wf2, bf2r,
                        Wf3, bf3r, NC)


# speedup vs baseline: 1.7069x; 1.1723x over previous
import functools

import jax
import jax.numpy as jnp
from jax import lax
from jax.experimental import pallas as pl
from jax.experimental.pallas import tpu as pltpu
from jax.experimental.pallas import tpu_sc as plsc

_CH = 125
_G = 64
_RB = 2000


def _fill_rows(ref, nrows, ncols, value):
    v = jnp.full((16,), value, dtype=jnp.float32)

    def body(j, c):
        for kk in range(ncols // 16):
            ref[j, pl.ds(kk * 16, 16)] = v
        return c

    lax.fori_loop(0, nrows, body, 0)


def _round_up(a, b):
    return -(-a // b) * b


@functools.cache
def _make_deg(N, E, NC, NS):
    NW = NC * NS
    NCH = E // (NW * _CH)
    RPT = _round_up(N, 16 * NS) // NS
    NPAD = RPT * NS
    mesh = plsc.VectorSubcoreMesh(core_axis_name="c", subcore_axis_name="s")

    @functools.partial(
        pl.kernel,
        out_type=jax.ShapeDtypeStruct((NC, NPAD, 16), jnp.float32),
        mesh=mesh,
        compiler_params=pltpu.CompilerParams(use_tc_tiling_on_sc=False),
        scratch_types=[
            pltpu.VMEM((NCH, _CH), jnp.int32),
            pltpu.VMEM((_CH, 16), jnp.float32),
            pltpu.VMEM((RPT, 16), jnp.float32),
            pltpu.SemaphoreType.DMA,
            pltpu.VMEM_SHARED((NPAD, 16), jnp.float32),
        ],
    )
    def k(e4_hbm, out_hbm, dstv, onesv, zbuf, ssem, acc):
        c = lax.axis_index("c")
        s = lax.axis_index("s")
        wid = s * NC + c
        _fill_rows(onesv, _CH, 16, 1.0)
        _fill_rows(zbuf, RPT, 16, 0.0)
        pltpu.sync_copy(zbuf, acc.at[pl.ds(s * RPT, RPT)])
        pltpu.sync_copy(e4_hbm.at[1, wid], dstv)
        plsc.subcore_barrier()

        def step(kk, carry):
            ds_ = [pltpu.async_copy(onesv, acc.at[dstv.at[8 * kk + b]],
                                    ssem, add=True) for b in range(8)]
            for d in ds_:
                d.wait()
            return carry

        lax.fori_loop(0, NCH // 8, step, 0)
        for j in range(NCH - NCH % 8, NCH):
            pltpu.sync_copy(onesv, acc.at[dstv.at[j]], add=True)
        plsc.subcore_barrier()
        pltpu.sync_copy(acc.at[pl.ds(s * RPT, RPT)],
                        out_hbm.at[c, pl.ds(s * RPT, RPT)])

    return k


@functools.cache
def _make_msg(N, E, D, NC, NS):
    NW = NC * NS
    NCH = E // (NW * _CH)
    RPT = _round_up(N, 16 * NS) // NS
    NPAD = RPT * NS
    mesh = plsc.VectorSubcoreMesh(core_axis_name="c", subcore_axis_name="s")

    KB = 4

    @functools.partial(
        pl.kernel,
        out_type=jax.ShapeDtypeStruct((NC, NPAD, D), jnp.bfloat16),
        mesh=mesh,
        compiler_params=pltpu.CompilerParams(use_tc_tiling_on_sc=False),
        scratch_types=[
            pltpu.VMEM((NCH, _CH), jnp.int32),
            pltpu.VMEM((NCH, _CH), jnp.int32),
            pltpu.VMEM((2 * KB, _CH, D), jnp.bfloat16),
            pltpu.SemaphoreType.DMA,
            pltpu.SemaphoreType.DMA,
            pltpu.SemaphoreType.DMA,
            pltpu.SemaphoreType.DMA,
            pltpu.VMEM_SHARED((NPAD, D), jnp.bfloat16),
        ],
    )
    def k(t_hbm, e4_hbm, out_hbm,
          srcv, dstv, rows, gsA, gsB, ssA, ssB, acc):
        c = lax.axis_index("c")
        s = lax.axis_index("s")
        wid = s * NC + c
        ZR = RPT // 8
        zv = jnp.zeros((32,), jnp.bfloat16)

        def zfill(j, carry):
            for kk in range(D // 32):
                rows[0, j, pl.ds(kk * 32, 32)] = zv
            return carry

        lax.fori_loop(0, ZR, zfill, 0)
        for t in range(8):
            pltpu.sync_copy(rows.at[0, pl.ds(0, ZR)],
                            acc.at[pl.ds(s * RPT + t * ZR, ZR)])
        pltpu.sync_copy(e4_hbm.at[0, wid], srcv)
        pltpu.sync_copy(e4_hbm.at[1, wid], dstv)
        plsc.subcore_barrier()

        def fire_gathers(bank, base, sem):
            for b in range(KB):
                pltpu.async_copy(t_hbm.at[srcv.at[base + b]],
                                 rows.at[bank * KB + b], sem)

        def wait_gathers(bank, sem):
            for b in range(KB):
                pltpu.make_async_copy(t_hbm.at[srcv.at[0]],
                                      rows.at[bank * KB + b], sem).wait()

        def scatter_bank(bank, base, sem):
            ds_ = [pltpu.async_copy(rows.at[bank * KB + b],
                                    acc.at[dstv.at[base + b]], sem, add=True)
                   for b in range(KB)]
            for d in ds_:
                d.wait()

        NCH_B = (NCH // (2 * KB)) * 2 * KB
        fire_gathers(0, 0, gsA)

        def step(kk, carry):
            base = 2 * KB * kk
            fire_gathers(1, base + KB, gsB)
            wait_gathers(0, gsA)
            scatter_bank(0, base, ssA)

            @pl.when(base + 3 * KB <= NCH_B)
            def _():
                fire_gathers(0, base + 2 * KB, gsA)

            wait_gathers(1, gsB)
            scatter_bank(1, base + KB, ssB)
            return carry

        lax.fori_loop(0, NCH // (2 * KB), step, 0)
        for j in range(NCH - NCH % (2 * KB), NCH):
            pltpu.async_copy(t_hbm.at[srcv.at[j]], rows.at[0], gsA).wait()
            pltpu.async_copy(rows.at[0], acc.at[dstv.at[j]], gsA,
                             add=True).wait()
        plsc.subcore_barrier()
        pltpu.sync_copy(acc.at[pl.ds(s * RPT, RPT)],
                        out_hbm.at[c, pl.ds(s * RPT, RPT)])

    return k


def _stage1a_call(x, W1):
    N, Din = x.shape
    D1 = W1.shape[1]
    NB = N // _RB

    def body(x_ref, w_ref, xw_ref):
        xw_ref[...] = jnp.dot(x_ref[...], w_ref[...],
                              preferred_element_type=jnp.float32)

    return pl.pallas_call(
        body,
        grid=(NB,),
        in_specs=[
            pl.BlockSpec((_RB, Din), lambda i: (i, 0)),
            pl.BlockSpec((Din, D1), lambda i: (0, 0)),
        ],
        out_specs=pl.BlockSpec((_RB, D1), lambda i: (i, 0)),
        out_shape=jax.ShapeDtypeStruct((N, D1), jnp.float32),
    )(x, W1)


def _stage1b_call(xw, degp, b1r, NC):
    N, D1 = xw.shape
    NB = N // _RB

    def body(xw_ref, dp_ref, b_ref, sc_ref, sb_ref, dis_ref, inv_ref):
        deg = 1.0 + sum(dp_ref[i, :, 0:1] for i in range(NC))
        dis = lax.rsqrt(deg)
        inv = 1.0 / deg
        xw = xw_ref[...]
        sc_ref[...] = (xw * dis).astype(jnp.bfloat16)
        sb_ref[...] = xw * inv + b_ref[...]
        dis_ref[...] = dis
        inv_ref[...] = inv

    return pl.pallas_call(
        body,
        grid=(NB,),
        in_specs=[
            pl.BlockSpec((_RB, D1), lambda i: (i, 0)),
            pl.BlockSpec((NC, _RB, 16), lambda i: (0, i, 0)),
            pl.BlockSpec((1, D1), lambda i: (0, 0)),
        ],
        out_specs=[
            pl.BlockSpec((_RB, D1), lambda i: (i, 0)),
            pl.BlockSpec((_RB, D1), lambda i: (i, 0)),
            pl.BlockSpec((_RB, 1), lambda i: (i, 0)),
            pl.BlockSpec((_RB, 1), lambda i: (i, 0)),
        ],
        out_shape=[
            jax.ShapeDtypeStruct((N, D1), jnp.bfloat16),
            jax.ShapeDtypeStruct((N, D1), jnp.float32),
            jax.ShapeDtypeStruct((N, 1), jnp.float32),
            jax.ShapeDtypeStruct((N, 1), jnp.float32),
        ],
    )(xw, degp, b1r)


def _stage2_call(acc1, dis, inv, selfb1, W2, b2r, NC):
    N = dis.shape[0]
    D1 = selfb1.shape[1]
    D2 = W2.shape[1]
    NB = N // _RB

    def body(a_ref, dis_ref, inv_ref, sb1_ref, w_ref, b_ref,
             sc_ref, sb2_ref):
        accsum = sum(a_ref[i].astype(jnp.float32) for i in range(NC))
        dis = dis_ref[...]
        h1 = jnp.maximum(dis * accsum + sb1_ref[...], 0.0)
        xw2 = jnp.dot(h1, w_ref[...], preferred_element_type=jnp.float32)
        sc_ref[...] = (xw2 * dis).astype(jnp.bfloat16)
        sb2_ref[...] = xw2 * inv_ref[...] + b_ref[...]

    return pl.pallas_call(
        body,
        grid=(NB,),
        in_specs=[
            pl.BlockSpec((NC, _RB, D1), lambda i: (0, i, 0)),
            pl.BlockSpec((_RB, 1), lambda i: (i, 0)),
            pl.BlockSpec((_RB, 1), lambda i: (i, 0)),
            pl.BlockSpec((_RB, D1), lambda i: (i, 0)),
            pl.BlockSpec((D1, D2), lambda i: (0, 0)),
            pl.BlockSpec((1, D2), lambda i: (0, 0)),
        ],
        out_specs=[
            pl.BlockSpec((_RB, D2), lambda i: (i, 0)),
            pl.BlockSpec((_RB, D2), lambda i: (i, 0)),
        ],
        out_shape=[
            jax.ShapeDtypeStruct((N, D2), jnp.bfloat16),
            jax.ShapeDtypeStruct((N, D2), jnp.float32),
        ],
    )(acc1, dis, inv, selfb1, W2, b2r)


def _stage3_call(acc2, dis, selfb2, batch_row, Wf1, bf1r, Wf2, bf2r,
                 Wf3, bf3r, NC):
    N = dis.shape[0]
    D2 = selfb2.shape[1]
    F1 = Wf1.shape[1]
    F2 = Wf2.shape[1]
    NB = N // _RB

    def body(a_ref, dis_ref, sb2_ref, b_ref, wf1_ref, bf1_ref,
             wf2_ref, bf2_ref, wf3_ref, bf3_ref, out_ref, sum_acc, cnt_acc):
        i = pl.program_id(0)
        accsum = sum(a_ref[k].astype(jnp.float32) for k in range(NC))
        h2 = jnp.maximum(dis_ref[...] * accsum + sb2_ref[...], 0.0)
        seg = b_ref[0]
        gids = lax.broadcasted_iota(jnp.int32, (_G, _RB), 0)
        pt = (gids == seg).astype(jnp.float32)
        part = jnp.dot(pt, h2, preferred_element_type=jnp.float32)
        cnt = jnp.dot(pt, jnp.ones((_RB, 1), jnp.float32),
                      preferred_element_type=jnp.float32)

        @pl.when(i == 0)
        def _():
            sum_acc[...] = part
            cnt_acc[...] = cnt

        @pl.when(i > 0)
        def _():
            sum_acc[...] += part
            cnt_acc[...] += cnt

        @pl.when(i == NB - 1)
        def _():
            pooled = sum_acc[...] / jnp.maximum(cnt_acc[...], 1.0)
            hh = jnp.maximum(
                jnp.dot(pooled, wf1_ref[...],
                        preferred_element_type=jnp.float32) + bf1_ref[...], 0.0)
            hh = jnp.maximum(
                jnp.dot(hh, wf2_ref[...],
                        preferred_element_type=jnp.float32) + bf2_ref[...], 0.0)
            out_ref[...] = (jnp.dot(hh, wf3_ref[...],
                                    preferred_element_type=jnp.float32)
                            + bf3_ref[...])

    return pl.pallas_call(
        body,
        grid=(NB,),
        in_specs=[
            pl.BlockSpec((NC, _RB, D2), lambda i: (0, i, 0)),
            pl.BlockSpec((_RB, 1), lambda i: (i, 0)),
            pl.BlockSpec((_RB, D2), lambda i: (i, 0)),
            pl.BlockSpec((1, 1, _RB), lambda i: (i, 0, 0)),
            pl.BlockSpec((D2, F1), lambda i: (0, 0)),
            pl.BlockSpec((1, F1), lambda i: (0, 0)),
            pl.BlockSpec((F1, F2), lambda i: (0, 0)),
            pl.BlockSpec((1, F2), lambda i: (0, 0)),
            pl.BlockSpec((F2, 1), lambda i: (0, 0)),
            pl.BlockSpec((1, 1), lambda i: (0, 0)),
        ],
        out_specs=pl.BlockSpec((_G, 1), lambda i: (0, 0)),
        out_shape=jax.ShapeDtypeStruct((_G, 1), jnp.float32),
        scratch_shapes=[
            pltpu.VMEM((_G, D2), jnp.float32),
            pltpu.VMEM((_G, 1), jnp.float32),
        ],
    )(acc2, dis, selfb2, batch_row, Wf1, bf1r, Wf2, bf2r, Wf3, bf3r)


def kernel(x, edge_index, batch, W1, b1, W2, b2, Wf1, bf1, Wf2, bf2, Wf3, bf3):
    N, Din = x.shape
    E = edge_index.shape[1]
    info = plsc.get_sparse_core_info()
    NC, NS = info.num_cores, info.num_subcores
    NW = NC * NS
    assert N % _RB == 0

    assert E % (NW * _CH) == 0
    e4 = edge_index.reshape(2, NW, E // (NW * _CH), _CH)
    batch_row = batch.reshape(N // _RB, 1, _RB)
    b1r = b1.reshape(1, -1)
    b2r = b2.reshape(1, -1)
    bf1r = bf1.reshape(1, -1)
    bf2r = bf2.reshape(1, -1)
    bf3r = bf3.reshape(1, -1)

    degp = _make_deg(N, E, NC, NS)(e4)
    xw1 = _stage1a_call(x, W1)
    scaled1, selfb1, dis, inv = _stage1b_call(xw1, degp, b1r, NC)
    acc1 = _make_msg(N, E, W1.shape[1], NC, NS)(scaled1, e4)
    scaled2, selfb2 = _stage2_call(acc1, dis, inv, selfb1, W2, b2r, NC)
    acc2 = _make_msg(N, E, W2.shape[1], NC, NS)(scaled2, e4)
    return _stage3_call(acc2, dis, selfb2, batch_row, Wf1, bf1r, Wf2, bf2r,
                        Wf3, bf3r, NC)
